# baseline (speedup 1.0000x reference)
import functools

import jax
import jax.numpy as jnp
from jax import lax
from jax.experimental import pallas as pl
from jax.experimental.pallas import tpu as pltpu
from jax.experimental.pallas import tpu_sc as plsc

D = 64
L = 16
NCORES = 2
NSUB = 16
NW = NCORES * NSUB
V = 1000000
BLK = 128
NFULL = V // BLK
TAIL_I0 = NFULL * BLK
TAIL_W = V - TAIL_I0
TMAIN = 244
NLEFT = NFULL - TMAIN * NW


def _mesh():
    return plsc.VectorSubcoreMesh(core_axis_name="c", subcore_axis_name="s")


def _wid():
    return lax.axis_index("s") * NCORES + lax.axis_index("c")


def _iota():
    return jnp.arange(L, dtype=jnp.int32)


_GDN = lax.GatherDimensionNumbers(
    offset_dims=(), collapsed_slice_dims=(0,), start_index_map=(0,))


def _shuffle16(s, idx):
    return lax.gather(
        s, idx.reshape(L, 1), _GDN, slice_sizes=(1,),
        mode=lax.GatherScatterMode.PROMISE_IN_BOUNDS)


def _lane_sum16(s):
    iota = _iota()
    for k in (8, 4, 2, 1):
        s = s + _shuffle16(s, iota ^ k)
    return s


def _rsqrt16(a):
    i = lax.bitcast_convert_type(a, jnp.int32)
    y = lax.bitcast_convert_type(
        jnp.int32(0x5F3759DF) - lax.shift_right_logical(i, 1), jnp.float32)
    for _ in range(3):
        y = y * (1.5 - 0.5 * a * y * y)
    return y


def _transpose_body(inb, outb, width):
    iota = _iota()
    rowi = [(iota + L * m) >> 1 for m in range(width // L)]
    coli = [(iota & 1) * D for _ in range(1)][0]

    def col(c, carry):
        cs = jnp.full((L,), c, jnp.int32) + coli
        for m in range(width // L):
            v = inb[c, pl.ds(L * m, L)]
            plsc.store_scatter(outb, [rowi[m], cs], v)
        return carry

    lax.fori_loop(0, D, col, 0)


def _make_transpose_kernel():
    @functools.partial(
        pl.kernel,
        mesh=_mesh(),
        compiler_params=pltpu.CompilerParams(needs_layout_passes=False),
        out_type=jax.ShapeDtypeStruct((V // 2, 2 * D), jnp.float32),
        scratch_types=[
            pltpu.VMEM((2, D, BLK), jnp.float32),
            pltpu.VMEM((2, BLK // 2, 2 * D), jnp.float32),
            pltpu.SemaphoreType.DMA,
            pltpu.SemaphoreType.DMA,
            pltpu.SemaphoreType.DMA,
            pltpu.SemaphoreType.DMA,
        ],
    )
    def k(tt_hbm, tail_hbm, tp_hbm, inb, outb, gi0, gi1, go0, go1):
        gin = [gi0, gi1]
        gout = [go0, go1]
        wid = _wid()

        def bid(t):
            return wid + NW * t

        def cin(t, b):
            return pltpu.make_async_copy(
                tt_hbm.at[:, pl.ds(BLK * bid(t), BLK)], inb.at[b], gin[b])

        def cout(t, b):
            return pltpu.make_async_copy(
                outb.at[b], tp_hbm.at[pl.ds((BLK // 2) * bid(t), BLK // 2), :],
                gout[b])

        for b in range(2):
            cin(b, b).start()

        def outer(g, carry):
            for b in range(2):
                t = 2 * g + b
                cin(t, b).wait()

                @pl.when(g >= 1)
                def _():
                    cout(t - 2, b).wait()

                _transpose_body(inb.at[b], outb.at[b], BLK)
                cout(t, b).start()

                @pl.when(g < TMAIN // 2 - 1)
                def _():
                    cin(t + 2, b).start()
            return carry

        lax.fori_loop(0, TMAIN // 2, outer, 0)
        for b in range(2):
            cout(TMAIN - 2 + b, b).wait()

        @pl.when(wid < NLEFT)
        def _():
            blid = TMAIN * NW + wid
            pltpu.sync_copy(tt_hbm.at[:, pl.ds(BLK * blid, BLK)], inb.at[0])
            _transpose_body(inb.at[0], outb.at[0], BLK)
            pltpu.sync_copy(
                outb.at[0],
                tp_hbm.at[pl.ds((BLK // 2) * blid, BLK // 2), :])

        @pl.when(wid == NLEFT)
        def _():
            pltpu.sync_copy(tail_hbm, inb.at[1])
            _transpose_body(inb.at[1], outb.at[1], TAIL_W)
            pltpu.sync_copy(
                outb.at[1, pl.ds(0, TAIL_W // 2), :],
                tp_hbm.at[pl.ds(TAIL_I0 // 2, TAIL_W // 2), :])

    return k


def _make_gather_kernel(nh, nb):
    ipw = nb // NW
    nk = ipw // BLK
    nchunk = nh * nk

    @functools.partial(
        pl.kernel,
        mesh=_mesh(),
        compiler_params=pltpu.CompilerParams(needs_layout_passes=False),
        out_type=jax.ShapeDtypeStruct((nh, D, nb), jnp.float32),
        scratch_types=[
            pltpu.VMEM((nh, ipw), jnp.int32),
            pltpu.VMEM((nh, ipw), jnp.int32),
            pltpu.VMEM((2, BLK, 2 * D), jnp.float32),
            pltpu.VMEM((2, D, BLK), jnp.float32),
            pltpu.SemaphoreType.DMA,
            pltpu.SemaphoreType.DMA,
            pltpu.SemaphoreType.DMA,
            pltpu.SemaphoreType.DMA,
        ],
    )
    def k(xt_hbm, tp_hbm, out_hbm, idx_v, idq_v, inb, stg,
          gi0, gi1, go0, go1):
        gin = [gi0, gi1]
        gout = [go0, go1]
        wid = _wid()
        base = ipw * wid
        pltpu.sync_copy(xt_hbm.at[:, pl.ds(base, ipw)], idx_v)

        def halve(h, carry):
            def chunk(t, carry2):
                v = idx_v[h, pl.ds(L * t, L)]
                idq_v[h, pl.ds(L * t, L)] = lax.shift_right_logical(v, 1)
                return carry2
            return lax.fori_loop(0, ipw // L, chunk, carry)

        lax.fori_loop(0, nh, halve, 0)

        iota = _iota()

        def hk(it):
            h = it // nk
            return h, it - nk * h

        def cin(it, b):
            h, kk = hk(it)
            return pltpu.make_async_copy(
                tp_hbm.at[idq_v.at[h, pl.ds(BLK * kk, BLK)]], inb.at[b],
                gin[b])

        def cout(it, b):
            h, kk = hk(it)
            return pltpu.make_async_copy(
                stg.at[b],
                out_hbm.at[h, :, pl.ds(base + BLK * kk, BLK)], gout[b])

        for b in range(2):
            cin(b, b).start()

        def group(m, carry, b, h, kk):
            rv = idx_v[h, pl.ds(BLK * kk + L * m, L)]
            offv = (rv & 1) * D
            rowsel = iota + L * m
            acc = None
            for c in range(D):
                val = plsc.load_gather(inb.at[b], [rowsel, offv + c])
                acc = val * val if acc is None else acc + val * val
            inv = _rsqrt16(acc)
            for c in range(D):
                val = plsc.load_gather(inb.at[b], [rowsel, offv + c])
                stg[b, c, pl.ds(L * m, L)] = val * inv
            return carry

        def outer(g, carry):
            for b in range(2):
                it = 2 * g + b
                h, kk = hk(it)
                cin(it, b).wait()

                @pl.when(g >= 1)
                def _():
                    cout(it - 2, b).wait()

                lax.fori_loop(0, BLK // L,
                              functools.partial(group, b=b, h=h, kk=kk), 0)
                cout(it, b).start()

                @pl.when(g < nchunk // 2 - 1)
                def _():
                    cin(it + 2, b).start()
            return carry

        lax.fori_loop(0, nchunk // 2, outer, 0)
        for b in range(2):
            cout(nchunk - 2 + b, b).wait()

    return k


def kernel(x, table):
    nb, nh = x.shape
    xt = jnp.transpose(x.astype(jnp.int32))
    tt = jnp.transpose(table)
    tail = jnp.pad(lax.slice(tt, (0, TAIL_I0), (D, V)),
                   ((0, 0), (0, BLK - TAIL_W)))
    tp = _make_transpose_kernel()(tt, tail)
    p = _make_gather_kernel(nh, nb)(xt, tp)
    return jnp.transpose(p, (2, 0, 1))


# speedup vs baseline: 4.5757x; 4.5757x over previous
import functools

import jax
import jax.numpy as jnp
from jax import lax
from jax.experimental import pallas as pl
from jax.experimental.pallas import tpu as pltpu
from jax.experimental.pallas import tpu_sc as plsc

D = 64
L = 16
NCORES = 2
NSUB = 16
NW = NCORES * NSUB
V = 1000000
BLK = 128
NFULL = V // BLK
TAIL_I0 = NFULL * BLK
TAIL_W = V - TAIL_I0
TMAIN = 244
NLEFT = NFULL - TMAIN * NW


def _mesh():
    return plsc.VectorSubcoreMesh(core_axis_name="c", subcore_axis_name="s")


def _wid():
    return lax.axis_index("s") * NCORES + lax.axis_index("c")


def _iota():
    return jnp.arange(L, dtype=jnp.int32)


def _rsqrt16(a):
    i = lax.bitcast_convert_type(a, jnp.int32)
    y = lax.bitcast_convert_type(
        jnp.int32(0x5F3759DF) - lax.shift_right_logical(i, 1), jnp.float32)
    for _ in range(3):
        y = y * (1.5 - 0.5 * a * y * y)
    return y


def _transpose_body(inb, outb, width):
    iota = _iota()

    @plsc.parallel_loop(0, L, unroll=2)
    def diag(d):
        perm = (iota + d) & (L - 1)
        rowb = lax.shift_right_logical(perm, 1)
        colp = (perm & 1) * D
        for u in range(width // L):
            src_i = perm + L * u
            dst_r = rowb + (L // 2) * u
            for ct in range(D // L):
                val = plsc.load_gather(inb, [iota + L * ct, src_i])
                plsc.store_scatter(outb, [dst_r, colp + (iota + L * ct)], val)


def _make_transpose_kernel():
    @functools.partial(
        pl.kernel,
        mesh=_mesh(),
        compiler_params=pltpu.CompilerParams(needs_layout_passes=False),
        out_type=jax.ShapeDtypeStruct((V // 2, 2 * D), jnp.float32),
        scratch_types=[
            pltpu.VMEM((2, D, BLK), jnp.float32),
            pltpu.VMEM((2, BLK // 2, 2 * D), jnp.float32),
            pltpu.SemaphoreType.DMA,
            pltpu.SemaphoreType.DMA,
            pltpu.SemaphoreType.DMA,
            pltpu.SemaphoreType.DMA,
        ],
    )
    def k(tt_hbm, tail_hbm, tp_hbm, inb, outb, gi0, gi1, go0, go1):
        gin = [gi0, gi1]
        gout = [go0, go1]
        wid = _wid()

        def bid(t):
            return wid + NW * t

        def cin(t, b):
            return pltpu.make_async_copy(
                tt_hbm.at[:, pl.ds(BLK * bid(t), BLK)], inb.at[b], gin[b])

        def cout(t, b):
            return pltpu.make_async_copy(
                outb.at[b],
                tp_hbm.at[pl.ds((BLK // 2) * bid(t), BLK // 2), :],
                gout[b])

        for b in range(2):
            cin(b, b).start()

        def outer(g, carry):
            for b in range(2):
                t = 2 * g + b
                cin(t, b).wait()

                @pl.when(g >= 1)
                def _():
                    cout(t - 2, b).wait()

                _transpose_body(inb.at[b], outb.at[b], BLK)
                cout(t, b).start()

                @pl.when(g < TMAIN // 2 - 1)
                def _():
                    cin(t + 2, b).start()
            return carry

        lax.fori_loop(0, TMAIN // 2, outer, 0)
        for b in range(2):
            cout(TMAIN - 2 + b, b).wait()

        @pl.when(wid < NLEFT)
        def _():
            blid = TMAIN * NW + wid
            pltpu.sync_copy(tt_hbm.at[:, pl.ds(BLK * blid, BLK)], inb.at[0])
            _transpose_body(inb.at[0], outb.at[0], BLK)
            pltpu.sync_copy(
                outb.at[0],
                tp_hbm.at[pl.ds((BLK // 2) * blid, BLK // 2), :])

        @pl.when(wid == NLEFT)
        def _():
            pltpu.sync_copy(tail_hbm, inb.at[1])
            _transpose_body(inb.at[1], outb.at[1], TAIL_W)
            pltpu.sync_copy(
                outb.at[1, pl.ds(0, TAIL_W // 2), :],
                tp_hbm.at[pl.ds(TAIL_I0 // 2, TAIL_W // 2), :])

    return k


def _make_gather_kernel(nh, nb):
    ipw = nb // NW
    nk = ipw // BLK
    nchunk = nh * nk

    @functools.partial(
        pl.kernel,
        mesh=_mesh(),
        compiler_params=pltpu.CompilerParams(needs_layout_passes=False),
        out_type=jax.ShapeDtypeStruct((nh, D, nb), jnp.float32),
        scratch_types=[
            pltpu.VMEM((nh, ipw), jnp.int32),
            pltpu.VMEM((nh, ipw), jnp.int32),
            pltpu.VMEM((2, BLK, 2 * D), jnp.float32),
            pltpu.VMEM((2, D, BLK), jnp.float32),
            pltpu.SemaphoreType.DMA,
            pltpu.SemaphoreType.DMA,
            pltpu.SemaphoreType.DMA,
            pltpu.SemaphoreType.DMA,
        ],
    )
    def k(xt_hbm, tp_hbm, out_hbm, idx_v, idq_v, inb, stg,
          gi0, gi1, go0, go1):
        gin = [gi0, gi1]
        gout = [go0, go1]
        wid = _wid()
        base = ipw * wid
        pltpu.sync_copy(xt_hbm.at[:, pl.ds(base, ipw)], idx_v)

        @plsc.parallel_loop(0, nh, unroll=2)
        def halve(h):
            for t in range(ipw // L):
                v = idx_v[h, pl.ds(L * t, L)]
                idq_v[h, pl.ds(L * t, L)] = lax.shift_right_logical(v, 1)

        iota = _iota()

        def hk(it):
            h = it // nk
            return h, it - nk * h

        def cin(it, b):
            h, kk = hk(it)
            return pltpu.make_async_copy(
                tp_hbm.at[idq_v.at[h, pl.ds(BLK * kk, BLK)]], inb.at[b],
                gin[b])

        def cout(it, b):
            h, kk = hk(it)
            return pltpu.make_async_copy(
                stg.at[b],
                out_hbm.at[h, :, pl.ds(base + BLK * kk, BLK)], gout[b])

        for b in range(2):
            cin(b, b).start()

        def run_groups(b, h, kk):
            @plsc.parallel_loop(0, BLK // L, unroll=2)
            def group(m):
                rv = idx_v[h, pl.ds(BLK * kk + L * m, L)]
                offv = (rv & 1) * D
                rowsel = iota + L * m
                accs = [None] * 4
                for c0 in range(D):
                    cv = (c0 + iota) & (D - 1)
                    val = plsc.load_gather(inb.at[b], [rowsel, offv + cv])
                    p = c0 & 3
                    accs[p] = (val * val if accs[p] is None
                               else accs[p] + val * val)
                inv = _rsqrt16((accs[0] + accs[1]) + (accs[2] + accs[3]))
                for c0 in range(D):
                    cv = (c0 + iota) & (D - 1)
                    val = plsc.load_gather(inb.at[b], [rowsel, offv + cv])
                    plsc.store_scatter(stg.at[b], [cv, rowsel], val * inv)

        def outer(g, carry):
            for b in range(2):
                it = 2 * g + b
                h, kk = hk(it)
                cin(it, b).wait()

                @pl.when(g >= 1)
                def _():
                    cout(it - 2, b).wait()

                run_groups(b, h, kk)
                cout(it, b).start()

                @pl.when(g < nchunk // 2 - 1)
                def _():
                    cin(it + 2, b).start()
            return carry

        lax.fori_loop(0, nchunk // 2, outer, 0)
        for b in range(2):
            cout(nchunk - 2 + b, b).wait()

    return k


def kernel(x, table):
    nb, nh = x.shape
    xt = jnp.transpose(x.astype(jnp.int32))
    tt = jnp.transpose(table)
    tail = jnp.pad(lax.slice(tt, (0, TAIL_I0), (D, V)),
                   ((0, 0), (0, BLK - TAIL_W)))
    tp = _make_transpose_kernel()(tt, tail)
    p = _make_gather_kernel(nh, nb)(xt, tp)
    return jnp.transpose(p, (2, 0, 1))
